# baseline (speedup 1.0000x reference)
import functools

import jax
import jax.numpy as jnp
from jax import lax
from jax.experimental import pallas as pl
from jax.experimental.pallas import tpu as pltpu
from jax.experimental.pallas import tpu_sc as plsc

N = 10000
E = 320000
D = 128
H = 128
C = 64
WD = 5e-4

NC = 2
NS = 16
NW = NC * NS
EPT = E // NW
CH = 80
NCHUNK = EPT // CH
NB = 3
NT = NCHUNK % NB
RA = 624
RTAIL = N - NS * RA
GRP = H // 32

_PI = [32 * (j // 32) + (2 * (j % 32) if j % 32 < 16 else 2 * (j % 32 - 16) + 1)
       for j in range(H)]
_PI2 = [_PI[_PI[j]] for j in range(H)]


def _make_spmm():
    mesh = plsc.VectorSubcoreMesh(core_axis_name="c", subcore_axis_name="s",
                                  num_cores=NC, num_subcores=NS)

    @functools.partial(
        pl.kernel,
        out_type=jax.ShapeDtypeStruct((NC, N, H), jnp.float32),
        mesh=mesh,
        scratch_types=[
            pltpu.VMEM((EPT,), jnp.int32),
            pltpu.VMEM((NB, CH, H), jnp.float32),
            pltpu.VMEM((NB, CH), jnp.int32),
            pltpu.VMEM((NB, CH), jnp.float32),
            pltpu.VMEM_SHARED((N, H), jnp.float32),
            [pltpu.SemaphoreType.DMA] * NB,
            [pltpu.SemaphoreType.DMA] * NB,
            [pltpu.SemaphoreType.DMA] * NB,
        ],
    )
    def spmm(h_hbm, src_hbm, dst_hbm, w_hbm, out_hbm,
             src_v, rows, sidx, wch, acc, gsem, isem, ssem):
        c = lax.axis_index("c")
        s = lax.axis_index("s")
        wid = c * NS + s
        e0 = wid * EPT

        cp0 = pltpu.async_copy(src_hbm.at[pl.ds(e0, EPT)], src_v, gsem[0])

        zv = jnp.zeros((16,), jnp.float32)

        def zrow(j, carry):
            for f in range(H // 16):
                rows[0, j, pl.ds(f * 16, 16)] = zv
            return carry

        lax.fori_loop(0, CH, zrow, 0)
        zcps = []
        for q in range(RA // CH):
            zcps.append(pltpu.async_copy(
                rows.at[0], acc.at[pl.ds(s * RA + q * CH, CH)], gsem[1]))
        zcps.append(pltpu.async_copy(
            rows.at[0].at[pl.ds(0, RA - (RA // CH) * CH)],
            acc.at[pl.ds(s * RA + (RA // CH) * CH, RA - (RA // CH) * CH)],
            gsem[1]))

        @pl.when(s == NS - 1)
        def _zero_tail():
            pltpu.async_copy(rows.at[0].at[pl.ds(0, RTAIL)],
                             acc.at[pl.ds(NS * RA, RTAIL)], gsem[2]).wait()

        for cp in zcps:
            cp.wait()
        cp0.wait()
        plsc.subcore_barrier()

        def prefetch(ci, b):
            off = pl.multiple_of(ci * CH, 8)
            pltpu.async_copy(dst_hbm.at[pl.ds(e0 + off, CH)], sidx.at[b], isem[b])
            pltpu.async_copy(w_hbm.at[pl.ds(e0 + off, CH)], wch.at[b], isem[b])
            pltpu.async_copy(h_hbm.at[src_v.at[pl.ds(off, CH)]], rows.at[b],
                             gsem[b])

        def wait_scatter(b):
            pltpu.make_async_copy(rows.at[b], acc.at[sidx.at[b]], ssem[b]).wait()

        def process(ci, b):
            off = pl.multiple_of(ci * CH, 8)
            pltpu.make_async_copy(dst_hbm.at[pl.ds(e0 + off, CH)], sidx.at[b],
                                  isem[b]).wait()
            pltpu.make_async_copy(w_hbm.at[pl.ds(e0 + off, CH)], wch.at[b],
                                  isem[b]).wait()
            pltpu.make_async_copy(h_hbm.at[src_v.at[pl.ds(off, CH)]],
                                  rows.at[b], gsem[b]).wait()

            def mrow(j16, inner):
                wv16 = wch[b, pl.ds(j16 * 16, 16)]
                for jj in range(16):
                    wv = jnp.broadcast_to(wv16[jj], (16,))
                    jr = j16 * 16 + jj
                    for f in range(H // 16):
                        sl = pl.ds(f * 16, 16)
                        rows[b, jr, sl] = rows[b, jr, sl] * wv
                return inner

            lax.fori_loop(0, CH // 16, mrow, 0)
            pltpu.async_copy(rows.at[b], acc.at[sidx.at[b]], ssem[b], add=True)

        prefetch(0, 0)
        prefetch(1, 1)

        def body(g, carry):
            for k in range(NB):
                ci = g * NB + k
                bp2 = (k + 2) % NB
                if k == 0:
                    @pl.when(g > 0)
                    def _w():
                        wait_scatter(bp2)
                else:
                    wait_scatter(bp2)
                prefetch(ci + 2, bp2)
                process(ci, k)
            return carry

        lax.fori_loop(0, NCHUNK // NB, body, 0)
        for t in range(NT):
            ci = (NCHUNK // NB) * NB + t
            wait_scatter((t + 2) % NB)
            process(ci, t)
        wait_scatter(NT - 1 if NT > 0 else NB - 1)

        plsc.subcore_barrier()
        pltpu.sync_copy(acc.at[pl.ds(s * RA, RA)],
                        out_hbm.at[c, pl.ds(s * RA, RA)])

        @pl.when(s == NS - 1)
        def _drain_tail():
            pltpu.sync_copy(acc.at[pl.ds(NS * RA, RTAIL)],
                            out_hbm.at[c, pl.ds(NS * RA, RTAIL)])

    return spmm


_spmm = _make_spmm()


def _mm1(x, W1):
    def k(x_ref, w_ref, o_ref):
        o_ref[...] = jnp.dot(x_ref[...], w_ref[...],
                             preferred_element_type=jnp.float32)

    return pl.pallas_call(
        k,
        grid=(10,),
        in_specs=[pl.BlockSpec((N // 10, D), lambda i: (i, 0)),
                  pl.BlockSpec((D, H), lambda i: (0, 0))],
        out_specs=pl.BlockSpec((N // 10, H), lambda i: (i, 0)),
        out_shape=jax.ShapeDtypeStruct((N, H), jnp.float32),
    )(x, W1)


def _relu_combine(p):
    def k(p_ref, o_ref):
        o_ref[...] = jnp.maximum(p_ref[0] + p_ref[1], 0.0)

    return pl.pallas_call(
        k,
        grid=(10,),
        in_specs=[pl.BlockSpec((2, N // 10, H), lambda i: (0, i, 0))],
        out_specs=pl.BlockSpec((N // 10, H), lambda i: (i, 0)),
        out_shape=jax.ShapeDtypeStruct((N, H), jnp.float32),
    )(p)


def _loss(u, W2p, label, mask_col, W1):
    def k(u_ref, w2_ref, lab_ref, msk_ref, w1_ref, o_ref):
        t = u_ref[0] + u_ref[1]
        logits = jnp.dot(t, w2_ref[...], preferred_element_type=jnp.float32)
        mx = jnp.max(logits, axis=1, keepdims=True)
        lse = jnp.log(jnp.sum(jnp.exp(logits - mx), axis=1, keepdims=True)) + mx
        logp = logits - lse
        li = -jnp.sum(lab_ref[...] * logp, axis=1)
        m = msk_ref[...][:, 0]
        mm = m / jnp.mean(m)
        ce = jnp.mean(li * mm)
        l2 = 0.5 * WD * jnp.sum(w1_ref[...] * w1_ref[...])
        o_ref[...] = jnp.reshape(ce + l2, (1, 1))

    out = pl.pallas_call(
        k,
        out_shape=jax.ShapeDtypeStruct((1, 1), jnp.float32),
    )(u, W2p, label, mask_col, W1)
    return out[0, 0]


def kernel(x, label, mask, edge_index, edge_weight, W1, W2):
    src = edge_index[0]
    dst = edge_index[1]
    h1 = _mm1(x, W1)
    p = _spmm(h1, src, dst, edge_weight)
    r = _relu_combine(p)
    u = _spmm(r, src, dst, edge_weight)
    return _loss(u, W2, label, mask.reshape(N, 1), W1)


# speedup vs baseline: 1.0002x; 1.0002x over previous
import functools

import jax
import jax.numpy as jnp
from jax import lax
from jax.experimental import pallas as pl
from jax.experimental.pallas import tpu as pltpu
from jax.experimental.pallas import tpu_sc as plsc

N = 10000
E = 320000
D = 128
H = 128
C = 64
WD = 5e-4

NC = 2
NS = 16
NW = NC * NS
EPT = E // NW
CH = 80
NCHUNK = EPT // CH
NB = 3
NT = NCHUNK % NB
RA = 624
RTAIL = N - NS * RA


def _make_spmm():
    mesh = plsc.VectorSubcoreMesh(core_axis_name="c", subcore_axis_name="s",
                                  num_cores=NC, num_subcores=NS)

    @functools.partial(
        pl.kernel,
        out_type=jax.ShapeDtypeStruct((NC, N, H), jnp.float32),
        mesh=mesh,
        scratch_types=[
            pltpu.VMEM((EPT,), jnp.int32),
            pltpu.VMEM((NB, CH, H), jnp.float32),
            pltpu.VMEM((NB, CH), jnp.int32),
            pltpu.VMEM((NB, CH), jnp.float32),
            pltpu.VMEM_SHARED((N, H), jnp.float32),
            [pltpu.SemaphoreType.DMA] * NB,
            [pltpu.SemaphoreType.DMA] * NB,
            [pltpu.SemaphoreType.DMA] * NB,
        ],
    )
    def spmm(h_hbm, src_hbm, dst_hbm, w_hbm, out_hbm,
             src_v, rows, sidx, wch, acc, gsem, isem, ssem):
        c = lax.axis_index("c")
        s = lax.axis_index("s")
        wid = c * NS + s
        e0 = wid * EPT

        cp0 = pltpu.async_copy(src_hbm.at[pl.ds(e0, EPT)], src_v, gsem[0])

        zv = jnp.zeros((16,), jnp.float32)

        def zrow(j, carry):
            for f in range(H // 16):
                rows[0, j, pl.ds(f * 16, 16)] = zv
            return carry

        lax.fori_loop(0, CH, zrow, 0)
        zcps = []
        for q in range(RA // CH):
            zcps.append(pltpu.async_copy(
                rows.at[0], acc.at[pl.ds(s * RA + q * CH, CH)], gsem[1]))
        zcps.append(pltpu.async_copy(
            rows.at[0].at[pl.ds(0, RA - (RA // CH) * CH)],
            acc.at[pl.ds(s * RA + (RA // CH) * CH, RA - (RA // CH) * CH)],
            gsem[1]))

        @pl.when(s == NS - 1)
        def _zero_tail():
            pltpu.async_copy(rows.at[0].at[pl.ds(0, RTAIL)],
                             acc.at[pl.ds(NS * RA, RTAIL)], gsem[2]).wait()

        for cp in zcps:
            cp.wait()
        cp0.wait()
        plsc.subcore_barrier()

        def prefetch(ci, b):
            off = pl.multiple_of(ci * CH, 8)
            pltpu.async_copy(dst_hbm.at[pl.ds(e0 + off, CH)], sidx.at[b], isem[b])
            pltpu.async_copy(w_hbm.at[pl.ds(e0 + off, CH)], wch.at[b], isem[b])
            pltpu.async_copy(h_hbm.at[src_v.at[pl.ds(off, CH)]], rows.at[b],
                             gsem[b])

        def wait_scatter(b):
            pltpu.make_async_copy(rows.at[b], acc.at[sidx.at[b]], ssem[b]).wait()

        def process(ci, b):
            off = pl.multiple_of(ci * CH, 8)
            pltpu.make_async_copy(dst_hbm.at[pl.ds(e0 + off, CH)], sidx.at[b],
                                  isem[b]).wait()
            pltpu.make_async_copy(w_hbm.at[pl.ds(e0 + off, CH)], wch.at[b],
                                  isem[b]).wait()
            pltpu.make_async_copy(h_hbm.at[src_v.at[pl.ds(off, CH)]],
                                  rows.at[b], gsem[b]).wait()

            def mrow(j16, inner):
                wv16 = wch[b, pl.ds(j16 * 16, 16)]
                for jj in range(16):
                    wv = jnp.broadcast_to(wv16[jj], (16,))
                    jr = j16 * 16 + jj
                    for f in range(H // 16):
                        sl = pl.ds(f * 16, 16)
                        rows[b, jr, sl] = rows[b, jr, sl] * wv
                return inner

            lax.fori_loop(0, CH // 16, mrow, 0)
            pltpu.async_copy(rows.at[b], acc.at[sidx.at[b]], ssem[b], add=True)

        prefetch(0, 0)
        prefetch(1, 1)

        def body(g, carry):
            for k in range(NB):
                ci = g * NB + k
                bp2 = (k + 2) % NB
                if k == 0:
                    @pl.when(g > 0)
                    def _w():
                        wait_scatter(bp2)
                else:
                    wait_scatter(bp2)
                prefetch(ci + 2, bp2)
                process(ci, k)
            return carry

        lax.fori_loop(0, NCHUNK // NB, body, 0)
        for t in range(NT):
            ci = (NCHUNK // NB) * NB + t
            wait_scatter((t + 2) % NB)
            process(ci, t)
        wait_scatter(NT - 1 if NT > 0 else NB - 1)

        plsc.subcore_barrier()
        pltpu.sync_copy(acc.at[pl.ds(s * RA, RA)],
                        out_hbm.at[c, pl.ds(s * RA, RA)])

        @pl.when(s == NS - 1)
        def _drain_tail():
            pltpu.sync_copy(acc.at[pl.ds(NS * RA, RTAIL)],
                            out_hbm.at[c, pl.ds(NS * RA, RTAIL)])

    return spmm


_spmm = _make_spmm()


def _mm1(x, W1):
    def k(x_ref, w_ref, o_ref):
        o_ref[...] = jnp.dot(x_ref[...], w_ref[...],
                             preferred_element_type=jnp.float32)

    return pl.pallas_call(
        k,
        grid=(10,),
        in_specs=[pl.BlockSpec((N // 10, D), lambda i: (i, 0)),
                  pl.BlockSpec((D, H), lambda i: (0, 0))],
        out_specs=pl.BlockSpec((N // 10, H), lambda i: (i, 0)),
        out_shape=jax.ShapeDtypeStruct((N, H), jnp.float32),
    )(x, W1)


def _relu_combine(p):
    def k(p_ref, o_ref):
        o_ref[...] = jnp.maximum(p_ref[0] + p_ref[1], 0.0)

    return pl.pallas_call(
        k,
        grid=(10,),
        in_specs=[pl.BlockSpec((2, N // 10, H), lambda i: (0, i, 0))],
        out_specs=pl.BlockSpec((N // 10, H), lambda i: (i, 0)),
        out_shape=jax.ShapeDtypeStruct((N, H), jnp.float32),
    )(p)


def _loss(u, W2p, label, mask_col, W1):
    def k(u_ref, w2_ref, lab_ref, msk_ref, w1_ref, o_ref):
        t = u_ref[0] + u_ref[1]
        logits = jnp.dot(t, w2_ref[...], preferred_element_type=jnp.float32)
        mx = jnp.max(logits, axis=1, keepdims=True)
        lse = jnp.log(jnp.sum(jnp.exp(logits - mx), axis=1, keepdims=True)) + mx
        logp = logits - lse
        li = -jnp.sum(lab_ref[...] * logp, axis=1)
        m = msk_ref[...][:, 0]
        mm = m / jnp.mean(m)
        ce = jnp.mean(li * mm)
        l2 = 0.5 * WD * jnp.sum(w1_ref[...] * w1_ref[...])
        o_ref[...] = jnp.reshape(ce + l2, (1, 1))

    out = pl.pallas_call(
        k,
        out_shape=jax.ShapeDtypeStruct((1, 1), jnp.float32),
    )(u, W2p, label, mask_col, W1)
    return out[0, 0]


def kernel(x, label, mask, edge_index, edge_weight, W1, W2):
    src = edge_index[0]
    dst = edge_index[1]
    h1 = _mm1(x, W1)
    p = _spmm(h1, src, dst, edge_weight)
    r = _relu_combine(p)
    u = _spmm(r, src, dst, edge_weight)
    return _loss(u, W2, label, mask.reshape(N, 1), W1)
